# Initial kernel scaffold; baseline (speedup 1.0000x reference)
import jax
import jax.numpy as jnp
from jax.experimental import pallas as pl


def kernel(x, w1, b1, w2, b2, wf1, bf1, wf2, bf2, hwa, gb, wn):
    raise NotImplementedError("write your pallas kernel here")


# speedup vs baseline: 19.0224x; 19.0224x over previous
import jax
import jax.numpy as jnp
from jax.experimental import pallas as pl
from jax.experimental.pallas import tpu as pltpu

N_CLASSES = 10
_S = 256
_PREC = jax.lax.Precision.HIGHEST


def _feat_kernel(x_ref, t1_ref, b1_ref, t2_ref, b2_ref,
                 wf1_ref, bf1_ref, wf2_ref, bf2_ref, hwa_ref, gb_ref, wn_ref,
                 o_ref, p1_ref, p2_ref):
    S = x_ref.shape[1]

    for c in range(7):
        acc = None
        for kh in range(5):
            lhs = x_ref[pl.ds(4 * c + kh, 4)].reshape(4 * S, 96)
            d = jnp.dot(lhs, t1_ref[kh], preferred_element_type=jnp.float32,
                        precision=_PREC)
            acc = d if acc is None else acc + d
        zz = acc.reshape(4, S, 112)
        for q in range(2):
            u = jnp.maximum(zz[2 * q], zz[2 * q + 1])
            v = jnp.maximum(u[:, :56], u[:, 56:])
            p1_ref[2 * c + q] = jnp.maximum(v + b1_ref[...], 0.0)

    for j in range(5):
        acc = None
        for kh in range(5):
            lhs = p1_ref[pl.ds(2 * j + kh, 2)].reshape(2 * S, 56)
            d = jnp.dot(lhs, t2_ref[kh], preferred_element_type=jnp.float32,
                        precision=_PREC)
            acc = d if acc is None else acc + d
        zz = acc.reshape(2, S, 120)
        u = jnp.maximum(zz[0], zz[1])
        v = jnp.maximum(u[:, :60], u[:, 60:])
        p2_ref[j] = jnp.maximum(v + b2_ref[...], 0.0)

    y = None
    for h in range(5):
        d = jnp.dot(p2_ref[h], wf1_ref[h], preferred_element_type=jnp.float32,
                    precision=_PREC)
        y = d if y is None else y + d
    y = jnp.maximum(y + bf1_ref[...], 0.0)

    f = jnp.dot(y, wf2_ref[...], preferred_element_type=jnp.float32,
                precision=_PREC) + bf2_ref[...]

    z = jnp.dot(f, hwa_ref[...], preferred_element_type=jnp.float32,
                precision=_PREC)
    xn = jnp.maximum(jnp.sqrt(jnp.sum(f * f, axis=-1, keepdims=True)), 1e-8)
    hcos = z[:, :N_CLASSES] / (xn * wn_ref[...])
    gl = z[:, N_CLASSES:N_CLASSES + 1] + gb_ref[...]
    o_ref[...] = jnp.concatenate([hcos, gl], axis=1)


def _head_kernel(a_ref, o_ref):
    a = a_ref[...]
    gl = a[:, N_CLASSES:N_CLASSES + 1]
    h = a[:, :N_CLASSES]
    mu = jnp.mean(gl, axis=0, keepdims=True)
    var = jnp.mean((gl - mu) ** 2, axis=0, keepdims=True)
    g = jax.nn.sigmoid((gl - mu) * jax.lax.rsqrt(var + 1e-5))
    out = g / h
    out = out - jnp.max(out, axis=-1, keepdims=True)
    e = jnp.exp(out)
    o_ref[...] = e / jnp.sum(e, axis=-1, keepdims=True)


@jax.jit
def _forward(x, w1, b1, w2, b2, wf1, bf1, wf2, bf2, hwa, gb, wn):
    B = x.shape[0]
    S = _S
    xr = jnp.transpose(x, (2, 0, 3, 1)).reshape(32, B, 96)
    t1 = jnp.transpose(w1, (0, 2, 1, 3)).reshape(5, 96, 112)
    t2 = jnp.transpose(w2, (0, 2, 1, 3)).reshape(5, 56, 120)

    part = pl.pallas_call(
        _feat_kernel,
        out_shape=jax.ShapeDtypeStruct((B, N_CLASSES + 1), jnp.float32),
        grid=(B // S,),
        in_specs=[
            pl.BlockSpec((32, S, 96), lambda i: (0, i, 0)),
            pl.BlockSpec((5, 96, 112), lambda i: (0, 0, 0)),
            pl.BlockSpec((1, 56), lambda i: (0, 0)),
            pl.BlockSpec((5, 56, 120), lambda i: (0, 0, 0)),
            pl.BlockSpec((1, 60), lambda i: (0, 0)),
            pl.BlockSpec((5, 60, 120), lambda i: (0, 0, 0)),
            pl.BlockSpec((1, 120), lambda i: (0, 0)),
            pl.BlockSpec((120, 64), lambda i: (0, 0)),
            pl.BlockSpec((1, 64), lambda i: (0, 0)),
            pl.BlockSpec((64, N_CLASSES + 1), lambda i: (0, 0)),
            pl.BlockSpec((1, 1), lambda i: (0, 0)),
            pl.BlockSpec((1, N_CLASSES), lambda i: (0, 0)),
        ],
        out_specs=pl.BlockSpec((S, N_CLASSES + 1), lambda i: (i, 0)),
        scratch_shapes=[pltpu.VMEM((14, S, 56), jnp.float32),
                        pltpu.VMEM((5, S, 60), jnp.float32)],
        compiler_params=pltpu.CompilerParams(
            dimension_semantics=("parallel",)),
    )(xr, t1, b1, t2, b2, wf1, bf1, wf2, bf2, hwa, gb, wn)

    pred = pl.pallas_call(
        _head_kernel,
        out_shape=jax.ShapeDtypeStruct((B, N_CLASSES), jnp.float32),
        grid=(1,),
        in_specs=[pl.BlockSpec((B, N_CLASSES + 1), lambda i: (0, 0))],
        out_specs=pl.BlockSpec((B, N_CLASSES), lambda i: (0, 0)),
        compiler_params=pltpu.CompilerParams(
            dimension_semantics=("arbitrary",)),
    )(part)
    return pred


def kernel(x, w1, b1, w2, b2, wf1, bf1, wf2, bf2, hwa, gb, wn):
    return _forward(x, w1, b1, w2, b2, wf1, bf1, wf2, bf2, hwa, gb, wn)


# speedup vs baseline: 22.3349x; 1.1741x over previous
import jax
import jax.numpy as jnp
from jax.experimental import pallas as pl
from jax.experimental.pallas import tpu as pltpu

N_CLASSES = 10
_S = 256


def _split(a):
    hi = a.astype(jnp.bfloat16)
    lo = (a - hi.astype(jnp.float32)).astype(jnp.bfloat16)
    return hi, lo


def _dot3(ah, al, bh, bl):
    d = lambda p, q: jnp.dot(p, q, preferred_element_type=jnp.float32)
    return d(ah, bh) + (d(al, bh) + d(ah, bl))


def _feat_kernel(xh_ref, xl_ref, t1_ref, b1_ref, t2_ref, b2_ref,
                 wf1_ref, bf1_ref, wf2_ref, bf2_ref, hwa_ref, gb_ref, wn_ref,
                 o_ref, p1h_ref, p1l_ref, p2h_ref, p2l_ref):
    S = xh_ref.shape[1]

    for c in range(7):
        acc = None
        for kh in range(5):
            lh = xh_ref[pl.ds(4 * c + kh, 4)].reshape(4 * S, 96)
            ll = xl_ref[pl.ds(4 * c + kh, 4)].reshape(4 * S, 96)
            d = _dot3(lh, ll, t1_ref[0, kh], t1_ref[1, kh])
            acc = d if acc is None else acc + d
        zz = acc.reshape(4, S, 112)
        for q in range(2):
            u = jnp.maximum(zz[2 * q], zz[2 * q + 1])
            v = jnp.maximum(u[:, :56], u[:, 56:])
            v = jnp.maximum(v + b1_ref[...], 0.0)
            vh, vl = _split(v)
            p1h_ref[2 * c + q] = vh
            p1l_ref[2 * c + q] = vl

    for j in range(5):
        acc = None
        for kh in range(5):
            lh = p1h_ref[pl.ds(2 * j + kh, 2)].reshape(2 * S, 56)
            ll = p1l_ref[pl.ds(2 * j + kh, 2)].reshape(2 * S, 56)
            d = _dot3(lh, ll, t2_ref[0, kh], t2_ref[1, kh])
            acc = d if acc is None else acc + d
        zz = acc.reshape(2, S, 120)
        u = jnp.maximum(zz[0], zz[1])
        v = jnp.maximum(u[:, :60], u[:, 60:])
        v = jnp.maximum(v + b2_ref[...], 0.0)
        vh, vl = _split(v)
        p2h_ref[j] = vh
        p2l_ref[j] = vl

    y = None
    for h in range(5):
        d = _dot3(p2h_ref[h], p2l_ref[h], wf1_ref[0, h], wf1_ref[1, h])
        y = d if y is None else y + d
    y = jnp.maximum(y + bf1_ref[...], 0.0)
    yh, yl = _split(y)

    f = _dot3(yh, yl, wf2_ref[0], wf2_ref[1]) + bf2_ref[...]
    fh, fl = _split(f)

    z = _dot3(fh, fl, hwa_ref[0], hwa_ref[1])
    xn = jnp.maximum(jnp.sqrt(jnp.sum(f * f, axis=-1, keepdims=True)), 1e-8)
    hcos = z[:, :N_CLASSES] / (xn * wn_ref[...])
    gl = z[:, N_CLASSES:N_CLASSES + 1] + gb_ref[...]
    o_ref[...] = jnp.concatenate([hcos, gl], axis=1)


def _head_kernel(a_ref, o_ref):
    a = a_ref[...]
    gl = a[:, N_CLASSES:N_CLASSES + 1]
    h = a[:, :N_CLASSES]
    mu = jnp.mean(gl, axis=0, keepdims=True)
    var = jnp.mean((gl - mu) ** 2, axis=0, keepdims=True)
    g = jax.nn.sigmoid((gl - mu) * jax.lax.rsqrt(var + 1e-5))
    out = g / h
    out = out - jnp.max(out, axis=-1, keepdims=True)
    e = jnp.exp(out)
    o_ref[...] = e / jnp.sum(e, axis=-1, keepdims=True)


def _split_stack(a):
    hi, lo = _split(a)
    return jnp.stack([hi, lo])


@jax.jit
def _forward(x, w1, b1, w2, b2, wf1, bf1, wf2, bf2, hwa, gb, wn):
    B = x.shape[0]
    S = _S
    xr = jnp.transpose(x, (2, 0, 3, 1)).reshape(32, B, 96)
    xh = xr.astype(jnp.bfloat16)
    xl = (xr - xh.astype(jnp.float32)).astype(jnp.bfloat16)
    t1 = _split_stack(jnp.transpose(w1, (0, 2, 1, 3)).reshape(5, 96, 112))
    t2 = _split_stack(jnp.transpose(w2, (0, 2, 1, 3)).reshape(5, 56, 120))
    wf1s = _split_stack(wf1)
    wf2s = _split_stack(wf2)
    hwas = _split_stack(hwa)

    part = pl.pallas_call(
        _feat_kernel,
        out_shape=jax.ShapeDtypeStruct((B, N_CLASSES + 1), jnp.float32),
        grid=(B // S,),
        in_specs=[
            pl.BlockSpec((32, S, 96), lambda i: (0, i, 0)),
            pl.BlockSpec((32, S, 96), lambda i: (0, i, 0)),
            pl.BlockSpec((2, 5, 96, 112), lambda i: (0, 0, 0, 0)),
            pl.BlockSpec((1, 56), lambda i: (0, 0)),
            pl.BlockSpec((2, 5, 56, 120), lambda i: (0, 0, 0, 0)),
            pl.BlockSpec((1, 60), lambda i: (0, 0)),
            pl.BlockSpec((2, 5, 60, 120), lambda i: (0, 0, 0, 0)),
            pl.BlockSpec((1, 120), lambda i: (0, 0)),
            pl.BlockSpec((2, 120, 64), lambda i: (0, 0, 0)),
            pl.BlockSpec((1, 64), lambda i: (0, 0)),
            pl.BlockSpec((2, 64, N_CLASSES + 1), lambda i: (0, 0, 0)),
            pl.BlockSpec((1, 1), lambda i: (0, 0)),
            pl.BlockSpec((1, N_CLASSES), lambda i: (0, 0)),
        ],
        out_specs=pl.BlockSpec((S, N_CLASSES + 1), lambda i: (i, 0)),
        scratch_shapes=[pltpu.VMEM((14, S, 56), jnp.bfloat16),
                        pltpu.VMEM((14, S, 56), jnp.bfloat16),
                        pltpu.VMEM((5, S, 60), jnp.bfloat16),
                        pltpu.VMEM((5, S, 60), jnp.bfloat16)],
        compiler_params=pltpu.CompilerParams(
            dimension_semantics=("parallel",)),
    )(xh, xl, t1, b1, t2, b2, wf1s, bf1, wf2s, bf2, hwas, gb, wn)

    pred = pl.pallas_call(
        _head_kernel,
        out_shape=jax.ShapeDtypeStruct((B, N_CLASSES), jnp.float32),
        grid=(1,),
        in_specs=[pl.BlockSpec((B, N_CLASSES + 1), lambda i: (0, 0))],
        out_specs=pl.BlockSpec((B, N_CLASSES), lambda i: (0, 0)),
        compiler_params=pltpu.CompilerParams(
            dimension_semantics=("arbitrary",)),
    )(part)
    return pred


def kernel(x, w1, b1, w2, b2, wf1, bf1, wf2, bf2, hwa, gb, wn):
    return _forward(x, w1, b1, w2, b2, wf1, bf1, wf2, bf2, hwa, gb, wn)
